# baseline (speedup 1.0000x reference)
import jax
import jax.numpy as jnp
from jax import lax
from jax.experimental import pallas as pl
from jax.experimental.pallas import tpu as pltpu
from jax.experimental.pallas import tpu_sc as plsc

BATCH = 16384
NC = 2
NS = 16
NW = NC * NS
LANES = 16
CHUNK = BATCH // NW

_C = (0.9999818714624722, -0.4991878401334513, 0.3244117606313534,
      -0.2086695713434938, 0.1002871370282292, -0.023689236277343366)


def _softplus(x):
    u = jnp.exp(-jnp.abs(x))
    p = _C[5]
    for c in (_C[4], _C[3], _C[2], _C[1], _C[0]):
        p = p * u + c
    return jnp.maximum(x, 0.0) + u * p


NP = 4
PIECE = CHUNK // NP


def _body(aidx_hbm, pidx_hbm, loss_hbm, tau_a_hbm, tau_p_hbm, out_hbm,
          aidx_v, pidx_v, loss_v, va_v, vp_v, out_v,
          sem_l, sem_o, *piece_sems):
    sem_i = piece_sems[:NP]
    sem_g = piece_sems[NP:]
    wid = lax.axis_index("s") * NC + lax.axis_index("c")
    base = wid * CHUNK

    cps = []
    for h in range(NP):
        off = base + h * PIECE
        dst = pl.ds(h * PIECE, PIECE)
        cps.append((pltpu.async_copy(aidx_hbm.at[pl.ds(off, PIECE)], aidx_v.at[dst], sem_i[h]),
                    pltpu.async_copy(pidx_hbm.at[pl.ds(off, PIECE)], pidx_v.at[dst], sem_i[h])))
    cp_l = pltpu.async_copy(loss_hbm.at[pl.ds(base, CHUNK)], loss_v, sem_l)

    gs = []
    for h in range(NP):
        cps[h][0].wait()
        cps[h][1].wait()
        sl = pl.ds(h * PIECE, PIECE)
        gs.append((pltpu.async_copy(tau_a_hbm.at[aidx_v.at[sl]], va_v.at[sl], sem_g[h]),
                   pltpu.async_copy(tau_p_hbm.at[pidx_v.at[sl]], vp_v.at[sl], sem_g[h])))

    cp_l.wait()
    for i in range(CHUNK // LANES):
        sl = pl.ds(i * LANES, LANES)
        loss_v[sl] = loss_v[sl] - 1.0

    outs = []
    for h in range(NP):
        gs[h][0].wait()
        gs[h][1].wait()
        for i in range(PIECE // LANES):
            sl = pl.ds(h * PIECE + i * LANES, LANES)
            out_v[sl] = loss_v[sl] * (_softplus(va_v[sl]) * _softplus(vp_v[sl]))
        outs.append(pltpu.async_copy(out_v.at[pl.ds(h * PIECE, PIECE)],
                                     out_hbm.at[pl.ds(base + h * PIECE, PIECE)], sem_o))
    for o in outs:
        o.wait()


@jax.jit
def kernel(amplitude_idxs, phase_idxs, losses, tau_amplitude, tau_phase):
    mesh = plsc.VectorSubcoreMesh(core_axis_name="c", subcore_axis_name="s")
    run = pl.kernel(
        _body,
        out_type=jax.ShapeDtypeStruct((BATCH,), jnp.float32),
        mesh=mesh,
        scratch_types=[
            pltpu.VMEM((CHUNK,), jnp.int32),
            pltpu.VMEM((CHUNK,), jnp.int32),
            pltpu.VMEM((CHUNK,), jnp.float32),
            pltpu.VMEM((CHUNK,), jnp.float32),
            pltpu.VMEM((CHUNK,), jnp.float32),
            pltpu.VMEM((CHUNK,), jnp.float32),
        ] + [pltpu.SemaphoreType.DMA] * (2 + 2 * NP),
    )
    return run(amplitude_idxs, phase_idxs, losses, tau_amplitude, tau_phase)


# speedup vs baseline: 1.0075x; 1.0075x over previous
import jax
import jax.numpy as jnp
from jax import lax
from jax.experimental import pallas as pl
from jax.experimental.pallas import tpu as pltpu
from jax.experimental.pallas import tpu_sc as plsc

BATCH = 16384
NC = 2
NS = 16
NW = NC * NS
LANES = 16
CHUNK = BATCH // NW

_C = (0.9999818714624722, -0.4991878401334513, 0.3244117606313534,
      -0.2086695713434938, 0.1002871370282292, -0.023689236277343366)


def _softplus(x):
    u = jnp.exp(-jnp.abs(x))
    p = _C[5]
    for c in (_C[4], _C[3], _C[2], _C[1], _C[0]):
        p = p * u + c
    return jnp.maximum(x, 0.0) + u * p


HALF = CHUNK // 2


def _body(aidx_hbm, pidx_hbm, loss_hbm, tau_a_hbm, tau_p_hbm, out_hbm,
          aidx_v, pidx_v, loss_v, va_v, vp_v, out_v,
          sem_a, sem_p, sem_l, sem_ga, sem_gp, sem_o):
    wid = lax.axis_index("s") * NC + lax.axis_index("c")
    base = wid * CHUNK

    cp_a = pltpu.async_copy(aidx_hbm.at[pl.ds(base, CHUNK)], aidx_v, sem_a)
    cp_p = pltpu.async_copy(pidx_hbm.at[pl.ds(base, CHUNK)], pidx_v, sem_p)
    cp_l = pltpu.async_copy(loss_hbm.at[pl.ds(base, CHUNK)], loss_v, sem_l)

    cp_a.wait()
    ga = pltpu.async_copy(tau_a_hbm.at[aidx_v], va_v, sem_ga)
    cp_p.wait()
    gp = pltpu.async_copy(tau_p_hbm.at[pidx_v], vp_v, sem_gp)
    cp_l.wait()

    for i in range(CHUNK // LANES):
        sl = pl.ds(i * LANES, LANES)
        loss_v[sl] = loss_v[sl] - 1.0

    ga.wait()
    gp.wait()
    outs = []
    for h in range(2):
        for i in range(HALF // LANES):
            sl = pl.ds(h * HALF + i * LANES, LANES)
            out_v[sl] = loss_v[sl] * (_softplus(va_v[sl]) * _softplus(vp_v[sl]))
        outs.append(pltpu.async_copy(out_v.at[pl.ds(h * HALF, HALF)],
                                     out_hbm.at[pl.ds(base + h * HALF, HALF)], sem_o))
    for o in outs:
        o.wait()


@jax.jit
def kernel(amplitude_idxs, phase_idxs, losses, tau_amplitude, tau_phase):
    mesh = plsc.VectorSubcoreMesh(core_axis_name="c", subcore_axis_name="s")
    run = pl.kernel(
        _body,
        out_type=jax.ShapeDtypeStruct((BATCH,), jnp.float32),
        mesh=mesh,
        scratch_types=[
            pltpu.VMEM((CHUNK,), jnp.int32),
            pltpu.VMEM((CHUNK,), jnp.int32),
            pltpu.VMEM((CHUNK,), jnp.float32),
            pltpu.VMEM((CHUNK,), jnp.float32),
            pltpu.VMEM((CHUNK,), jnp.float32),
            pltpu.VMEM((CHUNK,), jnp.float32),
        ] + [pltpu.SemaphoreType.DMA] * 6,
    )
    return run(amplitude_idxs, phase_idxs, losses, tau_amplitude, tau_phase)


# speedup vs baseline: 1.0113x; 1.0038x over previous
import jax
import jax.numpy as jnp
from jax import lax
from jax.experimental import pallas as pl
from jax.experimental.pallas import tpu as pltpu
from jax.experimental.pallas import tpu_sc as plsc

BATCH = 16384
NC = 2
NS = 16
NW = NC * NS
LANES = 16
CHUNK = BATCH // NW

_C = (0.9999818714624722, -0.4991878401334513, 0.3244117606313534,
      -0.2086695713434938, 0.1002871370282292, -0.023689236277343366)


def _softplus(x):
    u = jnp.exp(-jnp.abs(x))
    p = _C[5]
    for c in (_C[4], _C[3], _C[2], _C[1], _C[0]):
        p = p * u + c
    return jnp.maximum(x, 0.0) + u * p


HALF = CHUNK // 2


def _body(aidx_hbm, pidx_hbm, loss_hbm, tau_a_hbm, tau_p_hbm, out_hbm,
          aidx_v, pidx_v, loss_v, va_v, vp_v, out_v,
          sem_a, sem_p, sem_l, sem_ga, sem_gp, sem_o):
    wid = lax.axis_index("s") * NC + lax.axis_index("c")
    base = wid * CHUNK

    cp_a = pltpu.async_copy(aidx_hbm.at[pl.ds(base, CHUNK)], aidx_v, sem_a)
    cp_p = pltpu.async_copy(pidx_hbm.at[pl.ds(base, CHUNK)], pidx_v, sem_p)
    cp_l = pltpu.async_copy(loss_hbm.at[pl.ds(base, CHUNK)], loss_v, sem_l)

    cp_a.wait()
    ga = pltpu.async_copy(tau_a_hbm.at[aidx_v], va_v, sem_ga)
    cp_p.wait()
    gp = pltpu.async_copy(tau_p_hbm.at[pidx_v], vp_v, sem_gp)
    cp_l.wait()

    for i in range(CHUNK // LANES):
        sl = pl.ds(i * LANES, LANES)
        loss_v[sl] = loss_v[sl] - 1.0

    ga.wait()
    gp.wait()
    outs = []
    for h in range(2):
        for i in range(HALF // LANES):
            sl = pl.ds(h * HALF + i * LANES, LANES)
            out_v[sl] = loss_v[sl] * (_softplus(va_v[sl]) * _softplus(vp_v[sl]))
        outs.append(pltpu.async_copy(out_v.at[pl.ds(h * HALF, HALF)],
                                     out_hbm.at[pl.ds(base + h * HALF, HALF)], sem_o))
    for o in outs:
        o.wait()


@jax.jit
def kernel(amplitude_idxs, phase_idxs, losses, tau_amplitude, tau_phase):
    mesh = plsc.VectorSubcoreMesh(core_axis_name="c", subcore_axis_name="s")
    run = pl.kernel(
        _body,
        out_type=jax.ShapeDtypeStruct((BATCH,), jnp.float32),
        mesh=mesh,
        scratch_types=[
            pltpu.VMEM((CHUNK,), jnp.int32),
            pltpu.VMEM((CHUNK,), jnp.int32),
            pltpu.VMEM((CHUNK,), jnp.float32),
            pltpu.VMEM((CHUNK,), jnp.float32),
            pltpu.VMEM((CHUNK,), jnp.float32),
            pltpu.VMEM((CHUNK,), jnp.float32),
        ] + [pltpu.SemaphoreType.DMA] * 6,
        compiler_params=pltpu.CompilerParams(
            disable_bounds_checks=True,
            disable_semaphore_checks=True,
            skip_device_barrier=True,
        ),
    )
    return run(amplitude_idxs, phase_idxs, losses, tau_amplitude, tau_phase)


# speedup vs baseline: 1.0133x; 1.0020x over previous
import jax
import jax.numpy as jnp
from jax import lax
from jax.experimental import pallas as pl
from jax.experimental.pallas import tpu as pltpu
from jax.experimental.pallas import tpu_sc as plsc

BATCH = 16384
NC = 2
NS = 16
NW = NC * NS
LANES = 16
CHUNK = BATCH // NW

_C = (0.9999818714624722, -0.4991878401334513, 0.3244117606313534,
      -0.2086695713434938, 0.1002871370282292, -0.023689236277343366)


def _softplus(x):
    u = jnp.exp(-jnp.abs(x))
    p = _C[5]
    for c in (_C[4], _C[3], _C[2], _C[1], _C[0]):
        p = p * u + c
    return jnp.maximum(x, 0.0) + u * p


HALF = CHUNK // 2


def _body(aidx_hbm, pidx_hbm, loss_hbm, tau_a_hbm, tau_p_hbm, out_hbm,
          aidx_v, pidx_v, loss_v, va_v, vp_v, out_v,
          sem_a, sem_p, sem_l, sem_ga, sem_gp, sem_o):
    wid = lax.axis_index("s") * NC + lax.axis_index("c")
    base = wid * CHUNK

    cp_a = pltpu.async_copy(aidx_hbm.at[pl.ds(base, CHUNK)], aidx_v, sem_a)
    cp_p = pltpu.async_copy(pidx_hbm.at[pl.ds(base, CHUNK)], pidx_v, sem_p)
    cp_l = pltpu.async_copy(loss_hbm.at[pl.ds(base, CHUNK)], loss_v, sem_l)

    cp_a.wait()
    cp_p.wait()
    gs = []
    for h, sem_g in ((0, sem_ga), (1, sem_gp)):
        sl = pl.ds(h * HALF, HALF)
        gs.append((pltpu.async_copy(tau_a_hbm.at[aidx_v.at[sl]], va_v.at[sl], sem_g),
                   pltpu.async_copy(tau_p_hbm.at[pidx_v.at[sl]], vp_v.at[sl], sem_g)))
    cp_l.wait()

    for i in range(CHUNK // LANES):
        sl = pl.ds(i * LANES, LANES)
        loss_v[sl] = loss_v[sl] - 1.0

    outs = []
    for h in range(2):
        gs[h][0].wait()
        gs[h][1].wait()
        for i in range(HALF // LANES):
            sl = pl.ds(h * HALF + i * LANES, LANES)
            out_v[sl] = loss_v[sl] * (_softplus(va_v[sl]) * _softplus(vp_v[sl]))
        outs.append(pltpu.async_copy(out_v.at[pl.ds(h * HALF, HALF)],
                                     out_hbm.at[pl.ds(base + h * HALF, HALF)], sem_o))
    for o in outs:
        o.wait()


@jax.jit
def kernel(amplitude_idxs, phase_idxs, losses, tau_amplitude, tau_phase):
    mesh = plsc.VectorSubcoreMesh(core_axis_name="c", subcore_axis_name="s")
    run = pl.kernel(
        _body,
        out_type=jax.ShapeDtypeStruct((BATCH,), jnp.float32),
        mesh=mesh,
        scratch_types=[
            pltpu.VMEM((CHUNK,), jnp.int32),
            pltpu.VMEM((CHUNK,), jnp.int32),
            pltpu.VMEM((CHUNK,), jnp.float32),
            pltpu.VMEM((CHUNK,), jnp.float32),
            pltpu.VMEM((CHUNK,), jnp.float32),
            pltpu.VMEM((CHUNK,), jnp.float32),
        ] + [pltpu.SemaphoreType.DMA] * 6,
        compiler_params=pltpu.CompilerParams(
            disable_bounds_checks=True,
            disable_semaphore_checks=True,
            skip_device_barrier=True,
        ),
    )
    return run(amplitude_idxs, phase_idxs, losses, tau_amplitude, tau_phase)
